# Initial kernel scaffold; baseline (speedup 1.0000x reference)
import jax
import jax.numpy as jnp
from jax.experimental import pallas as pl


def kernel(selector_params, component_params):
    raise NotImplementedError("write your pallas kernel here")


# speedup vs baseline: 1.2385x; 1.2385x over previous
import functools

import jax
import jax.numpy as jnp
from jax import lax
from jax.experimental import pallas as pl
from jax.experimental.pallas import tpu as pltpu
from jax.experimental.pallas import tpu_sc as plsc

N_TOKENS = 8192
N_EXPERTS = 64
N_CATEGORIES = 2048

_NC, _NS = 2, 16
NW = _NC * _NS
BPW = N_TOKENS // NW
CH = 16
NCH = BPW // CH


def _prep_body(sel_ref, comp_ref, idx_ref, p_ref):
    x = sel_ref[...]
    m = jnp.max(x, axis=-1, keepdims=True)
    ii = lax.broadcasted_iota(jnp.int32, x.shape, 1)
    idx_ref[...] = jnp.min(jnp.where(x == m, ii, N_EXPERTS), axis=-1)
    c = comp_ref[...]
    e = jnp.exp(c - jnp.max(c, axis=-1, keepdims=True))
    p_ref[...] = e / jnp.sum(e, axis=-1, keepdims=True)


def _prep(selector_params, component_params):
    return pl.pallas_call(
        _prep_body,
        out_shape=[
            jax.ShapeDtypeStruct((N_TOKENS,), jnp.int32),
            jax.ShapeDtypeStruct((N_EXPERTS, N_CATEGORIES), jnp.float32),
        ],
    )(selector_params, component_params)


@functools.cache
def _make_gather_sc():
    mesh = plsc.VectorSubcoreMesh(
        core_axis_name="c", subcore_axis_name="s",
        num_cores=_NC, num_subcores=_NS)

    @functools.partial(
        pl.kernel,
        out_type=jax.ShapeDtypeStruct((N_TOKENS, N_CATEGORIES), jnp.float32),
        mesh=mesh,
        scratch_types=[
            pltpu.VMEM((NCH, CH), jnp.int32),
            pltpu.VMEM((CH, N_CATEGORIES), jnp.float32),
            pltpu.SemaphoreType.DMA,
        ],
    )
    def _gather_sc(p_hbm, idx_hbm, out_hbm, idx_v, buf, gsem):
        wid = lax.axis_index("s") * _NC + lax.axis_index("c")
        base = wid * BPW
        pltpu.sync_copy(idx_hbm.at[wid], idx_v)

        def body(g, carry):
            pltpu.async_copy(p_hbm.at[idx_v.at[g]], buf, gsem).wait()
            pltpu.sync_copy(buf, out_hbm.at[pl.ds(base + g * CH, CH)])
            return carry

        lax.fori_loop(0, NCH, body, 0)

    return _gather_sc


def kernel(selector_params, component_params):
    idx, p = _prep(selector_params, component_params)
    return _make_gather_sc()(p, idx.reshape(NW, NCH, CH))


# speedup vs baseline: 1.2744x; 1.0290x over previous
import functools

import jax
import jax.numpy as jnp
from jax import lax
from jax.experimental import pallas as pl
from jax.experimental.pallas import tpu as pltpu
from jax.experimental.pallas import tpu_sc as plsc

N_TOKENS = 8192
N_EXPERTS = 64
N_CATEGORIES = 2048

_NC, _NS = 2, 16
NW = _NC * _NS
BPW = N_TOKENS // NW
CH = 16
NCH = BPW // CH


def _prep_body(sel_ref, comp_ref, idx_ref, p_ref):
    x = sel_ref[...]
    m = jnp.max(x, axis=-1, keepdims=True)
    ii = lax.broadcasted_iota(jnp.int32, x.shape, 1)
    idx_ref[...] = jnp.min(jnp.where(x == m, ii, N_EXPERTS), axis=-1)
    c = comp_ref[...]
    e = jnp.exp(c - jnp.max(c, axis=-1, keepdims=True))
    p_ref[...] = e / jnp.sum(e, axis=-1, keepdims=True)


def _prep(selector_params, component_params):
    return pl.pallas_call(
        _prep_body,
        out_shape=[
            jax.ShapeDtypeStruct((N_TOKENS,), jnp.int32),
            jax.ShapeDtypeStruct((N_EXPERTS, N_CATEGORIES), jnp.float32),
        ],
    )(selector_params, component_params)


@functools.cache
def _make_gather_sc():
    mesh = plsc.VectorSubcoreMesh(
        core_axis_name="c", subcore_axis_name="s",
        num_cores=_NC, num_subcores=_NS)

    @functools.partial(
        pl.kernel,
        out_type=jax.ShapeDtypeStruct((N_TOKENS, N_CATEGORIES), jnp.float32),
        mesh=mesh,
        scratch_types=[
            pltpu.VMEM((NCH, CH), jnp.int32),
            pltpu.VMEM((CH, N_CATEGORIES), jnp.float32),
            pltpu.VMEM((CH, N_CATEGORIES), jnp.float32),
            pltpu.SemaphoreType.DMA,
            pltpu.SemaphoreType.DMA,
            pltpu.SemaphoreType.DMA,
            pltpu.SemaphoreType.DMA,
        ],
    )
    def _gather_sc(p_hbm, idx_hbm, out_hbm, idx_v, buf0, buf1, gs0, gs1,
                   ss0, ss1):
        wid = lax.axis_index("s") * _NC + lax.axis_index("c")
        base = wid * BPW
        pltpu.sync_copy(idx_hbm.at[wid], idx_v)
        bufs = ((buf0, gs0, ss0), (buf1, gs1, ss1))

        def gather(c, buf, gs):
            pltpu.async_copy(p_hbm.at[idx_v.at[c]], buf, gs)

        def wait_gather(c, buf, gs):
            pltpu.make_async_copy(p_hbm.at[idx_v.at[c]], buf, gs).wait()

        def scatter(c, buf, ss):
            pltpu.async_copy(buf, out_hbm.at[pl.ds(base + c * CH, CH)], ss)

        def wait_scatter(c, buf, ss):
            pltpu.make_async_copy(
                buf, out_hbm.at[pl.ds(base + c * CH, CH)], ss).wait()

        gather(0, buf0, gs0)
        gather(1, buf1, gs1)

        def body(i, carry):
            for b, (buf, gs, ss) in enumerate(bufs):
                c = 2 * i + b
                wait_gather(c, buf, gs)
                scatter(c, buf, ss)
                wait_scatter(c, buf, ss)
                gather(c + 2, buf, gs)
            return carry

        lax.fori_loop(0, NCH // 2 - 1, body, 0)
        for b, (buf, gs, ss) in enumerate(bufs):
            c = NCH - 2 + b
            wait_gather(c, buf, gs)
            scatter(c, buf, ss)
            wait_scatter(c, buf, ss)

    return _gather_sc


def kernel(selector_params, component_params):
    idx, p = _prep(selector_params, component_params)
    return _make_gather_sc()(p, idx.reshape(NW, NCH, CH))


# speedup vs baseline: 1.8232x; 1.4307x over previous
import functools

import jax
import jax.numpy as jnp
from jax import lax
from jax.experimental import pallas as pl
from jax.experimental.pallas import tpu as pltpu
from jax.experimental.pallas import tpu_sc as plsc

N_TOKENS = 8192
N_EXPERTS = 64
N_CATEGORIES = 2048

_NC, _NS = 2, 16
NW = _NC * _NS
BPW = N_TOKENS // NW
CH = 16
NCH = BPW // CH


def _prep_body(sel_ref, comp_ref, idx_ref, p_ref):
    x = sel_ref[...]
    m = jnp.max(x, axis=-1, keepdims=True)
    ii = lax.broadcasted_iota(jnp.int32, x.shape, 1)
    idx_ref[...] = jnp.min(jnp.where(x == m, ii, N_EXPERTS), axis=-1)
    c = comp_ref[...]
    e = jnp.exp(c - jnp.max(c, axis=-1, keepdims=True))
    p_ref[...] = e / jnp.sum(e, axis=-1, keepdims=True)


def _prep(selector_params, component_params):
    return pl.pallas_call(
        _prep_body,
        out_shape=[
            jax.ShapeDtypeStruct((N_TOKENS,), jnp.int32),
            jax.ShapeDtypeStruct((N_EXPERTS, N_CATEGORIES), jnp.float32),
        ],
    )(selector_params, component_params)


@functools.cache
def _make_gather_sc():
    mesh = plsc.VectorSubcoreMesh(
        core_axis_name="c", subcore_axis_name="s",
        num_cores=_NC, num_subcores=_NS)

    @functools.partial(
        pl.kernel,
        out_type=jax.ShapeDtypeStruct((N_TOKENS, N_CATEGORIES), jnp.float32),
        mesh=mesh,
        scratch_types=[
            pltpu.VMEM_SHARED((N_EXPERTS, N_CATEGORIES), jnp.float32),
            pltpu.VMEM((NCH, CH), jnp.int32),
            pltpu.VMEM((CH, N_CATEGORIES), jnp.float32),
            pltpu.VMEM((CH, N_CATEGORIES), jnp.float32),
            pltpu.SemaphoreType.DMA,
            pltpu.SemaphoreType.DMA,
            pltpu.SemaphoreType.DMA,
            pltpu.SemaphoreType.DMA,
        ],
    )
    def _gather_sc(p_hbm, idx_hbm, out_hbm, p_sh, idx_v, buf0,
                   buf1, gs0, gs1, ss0, ss1):
        sid = lax.axis_index("s")
        wid = sid * _NC + lax.axis_index("c")
        base = wid * BPW
        @pl.when(sid == 0)
        def _():
            pltpu.sync_copy(p_hbm, p_sh)

        pltpu.sync_copy(idx_hbm.at[wid], idx_v)
        plsc.subcore_barrier()
        bufs = ((buf0, gs0, ss0), (buf1, gs1, ss1))

        def gather(c, buf, gs):
            ev = idx_v[c]
            for j in range(CH):
                e = ev[j]
                pltpu.async_copy(p_sh.at[e], buf.at[j], gs)

        def wait_gather(c, buf, gs):
            pltpu.make_async_copy(p_sh.at[pl.ds(0, CH)], buf, gs).wait()

        def scatter(c, buf, ss):
            pltpu.async_copy(buf, out_hbm.at[pl.ds(base + c * CH, CH)], ss)

        def wait_scatter(c, buf, ss):
            pltpu.make_async_copy(
                buf, out_hbm.at[pl.ds(base + c * CH, CH)], ss).wait()

        gather(0, buf0, gs0)
        gather(1, buf1, gs1)

        def body(i, carry):
            for b, (buf, gs, ss) in enumerate(bufs):
                c = 2 * i + b
                wait_gather(c, buf, gs)
                scatter(c, buf, ss)
                wait_scatter(c, buf, ss)
                gather(c + 2, buf, gs)
            return carry

        lax.fori_loop(0, NCH // 2 - 1, body, 0)
        for b, (buf, gs, ss) in enumerate(bufs):
            c = NCH - 2 + b
            wait_gather(c, buf, gs)
            scatter(c, buf, ss)
            wait_scatter(c, buf, ss)

    return _gather_sc


def kernel(selector_params, component_params):
    idx, p = _prep(selector_params, component_params)
    return _make_gather_sc()(p, idx.reshape(NW, NCH, CH))
